# baseline (speedup 1.0000x reference)
import functools

import jax
import jax.numpy as jnp
from jax import lax
from jax.experimental import pallas as pl
from jax.experimental.pallas import tpu as pltpu
from jax.experimental.pallas import tpu_sc as plsc

NC = 2
NS = 16
LANES = 16
NW = NC * NS

L = 8
C = 16

CH = 2048
TB = 8192


def _newton_rsqrt(ss):
    i = lax.bitcast_convert_type(ss, jnp.int32)
    y = lax.bitcast_convert_type(0x5F3759DF - (i >> 1), jnp.float32)
    for _ in range(3):
        y = y * (1.5 - 0.5 * ss * y * y)
    return jnp.where(ss > 0.0, y, 0.0)


def _sc_pass(pred2, gt_flat, n_per_batch, vpw):
    k_chunks = vpw // CH
    mesh = plsc.VectorSubcoreMesh(core_axis_name="c", subcore_axis_name="s",
                                  num_cores=NC, num_subcores=NS)

    @functools.partial(
        pl.kernel,
        out_type=jax.ShapeDtypeStruct((NW, L, C * LANES), jnp.float32),
        mesh=mesh,
        compiler_params=pltpu.CompilerParams(needs_layout_passes=False),
        scratch_types=[
            pltpu.VMEM((2, C, CH), jnp.float32),
            pltpu.VMEM((2, CH), jnp.int32),
            pltpu.VMEM((L, C * LANES), jnp.float32),
            pltpu.SemaphoreType.DMA,
            pltpu.SemaphoreType.DMA,
            pltpu.SemaphoreType.DMA,
            pltpu.SemaphoreType.DMA,
        ],
    )
    def kern(pred_hbm, gt_hbm, nsum_out,
             buf, labv, nsum_t, sp0, sp1, sl0, sl1):
        cid = lax.axis_index("c")
        sid = lax.axis_index("s")
        wid = sid * NC + cid
        batch = wid // NS
        slot = wid % NS
        row0 = batch * C
        col0 = slot * vpw

        psem = (sp0, sp1)
        lsem = (sl0, sl1)

        zero16 = jnp.zeros((LANES,), jnp.float32)
        lane = lax.iota(jnp.int32, 16)
        idx1 = [lane + c * LANES for c in range(C)]

        def zinit_row(i, _):
            r = i // C
            j = i % C
            nsum_t[r, pl.ds(j * LANES, LANES)] = zero16
            return 0
        lax.fori_loop(0, L * C, zinit_row, 0)

        def chunk_coff(k):
            return col0 + k * CH

        def start(k, b):
            coff = chunk_coff(k)
            pltpu.async_copy(
                pred_hbm.at[pl.ds(row0, C), pl.ds(coff, CH)], buf.at[b],
                psem[b])
            pltpu.async_copy(
                gt_hbm.at[pl.ds(batch * n_per_batch + coff, CH)], labv.at[b],
                lsem[b])

        def wait(k, b):
            coff = chunk_coff(k)
            pltpu.make_async_copy(
                pred_hbm.at[pl.ds(row0, C), pl.ds(coff, CH)], buf.at[b],
                psem[b]).wait()
            pltpu.make_async_copy(
                gt_hbm.at[pl.ds(batch * n_per_batch + coff, CH)], labv.at[b],
                lsem[b]).wait()

        def compute(b):
            @plsc.parallel_loop(0, CH // LANES, unroll=3)
            def grp(g):
                base = g * LANES
                lv = labv[b, pl.ds(base, LANES)]
                vs = []
                sq = []
                for c in range(C):
                    v = buf[b, c, pl.ds(base, LANES)]
                    vs.append(v)
                    sq.append(v * v)
                while len(sq) > 1:
                    sq = [sq[i] + sq[i + 1] for i in range(0, len(sq), 2)]
                rinv = _newton_rsqrt(sq[0])
                for c in range(C):
                    plsc.addupdate_scatter(nsum_t, [lv, idx1[c]],
                                           vs[c] * rinv)

        start(0, 0)

        def pair_body(k2, _):
            k = k2 * 2
            start(k + 1, 1)
            wait(k, 0)
            compute(0)

            @pl.when(k2 < n_pairs - 1)
            def _():
                start(k + 2, 0)

            wait(k + 1, 1)
            compute(1)
            return 0

        n_pairs = k_chunks // 2
        lax.fori_loop(0, n_pairs, pair_body, 0)

        pltpu.sync_copy(nsum_t, nsum_out.at[wid])

    return kern(pred2, gt_flat)


def _tc_sums_body(pred_ref, gt_ref, sums_ref, cnt_ref):
    pid = pl.program_id(0)

    @pl.when(pid == 0)
    def _():
        sums_ref[...] = jnp.zeros_like(sums_ref)
        cnt_ref[...] = jnp.zeros_like(cnt_ref)

    gt_blk = gt_ref[...]
    l_iota = lax.broadcasted_iota(jnp.int32, (L, TB), 0)
    oh0 = (gt_blk[0:1, :] == l_iota).astype(jnp.float32)
    oh1 = (gt_blk[1:2, :] == l_iota).astype(jnp.float32)
    p0 = pred_ref[0:C, :]
    p1 = pred_ref[C:2 * C, :]
    dn = (((1,), (1,)), ((), ()))
    s = (lax.dot_general(oh0, p0, dn, preferred_element_type=jnp.float32)
         + lax.dot_general(oh1, p1, dn,
                           preferred_element_type=jnp.float32))
    sums_ref[...] += s
    cnt = (jnp.sum(oh0, axis=1, keepdims=True)
           + jnp.sum(oh1, axis=1, keepdims=True))
    cnt_ref[...] += jnp.broadcast_to(cnt, (L, C))


def _finalize_body(cnt_ref, sum_ref, nsum_ref, out_ref):
    rows = NW * L
    lmat = lax.broadcasted_iota(jnp.int32, (L, rows), 0)
    jmat = lax.broadcasted_iota(jnp.int32, (L, rows), 1)
    sel = (jmat % L == lmat).astype(jnp.float32)

    j2 = lax.broadcasted_iota(jnp.int32, (C * LANES, C), 0)
    c2 = lax.broadcasted_iota(jnp.int32, (C * LANES, C), 1)
    red = (j2 // LANES == c2).astype(jnp.float32)

    counts = cnt_ref[:, 0:1]
    sums = sum_ref[...]
    nsums = jnp.dot(jnp.dot(sel, nsum_ref[...],
                            preferred_element_type=jnp.float32),
                    red, preferred_element_type=jnp.float32)

    safe_c = jnp.maximum(counts, 1.0)
    means = sums / safe_c

    snorm = jnp.sqrt(jnp.sum(sums * sums, axis=1, keepdims=True))
    cos_sum = jnp.sum(sums * nsums, axis=1, keepdims=True) / jnp.maximum(
        snorm, 1e-30)
    intra_per_label = cos_sum / safe_c

    lab_idx = lax.broadcasted_iota(jnp.int32, (L, 1), 0)
    nonbg = (lab_idx > 0).astype(jnp.float32)
    intra_sim = jnp.sum(intra_per_label * nonbg, keepdims=True) / (L - 1.0)

    mnorm = jnp.sqrt(jnp.sum(means * means, axis=1, keepdims=True))
    mn = means / jnp.maximum(mnorm, 1e-8)

    total = jnp.zeros((1, 1), jnp.float32)
    for i in range(1, L - 1):
        row_i = mn[i:i + 1, :]
        simr = jnp.sum(mn * row_i, axis=1, keepdims=True)
        pair = (lab_idx > i).astype(jnp.float32)
        total = total + jnp.sum(jnp.clip(simr, 0.0, 1.0) * pair,
                                keepdims=True)
    n_pairs = (L - 1) * (L - 2) // 2
    inter = total / float(n_pairs)

    out_ref[...] = inter - intra_sim


def kernel(prediction, gt):
    b, c, z, y, x = prediction.shape
    n_per_batch = z * y * x
    n_total = b * n_per_batch
    vpw = n_per_batch // NS

    pred2 = prediction.reshape(b * c, n_per_batch)
    gt_flat = gt.reshape(n_total)
    gt2 = gt.reshape(b, n_per_batch)

    nsum_p = _sc_pass(pred2, gt_flat, n_per_batch, vpw)

    grid = n_per_batch // TB
    sums_tc, cnt_tc = pl.pallas_call(
        _tc_sums_body,
        grid=(grid,),
        in_specs=[
            pl.BlockSpec((b * c, TB), lambda i: (0, i)),
            pl.BlockSpec((b, TB), lambda i: (0, i)),
        ],
        out_specs=[
            pl.BlockSpec((L, C), lambda i: (0, 0)),
            pl.BlockSpec((L, C), lambda i: (0, 0)),
        ],
        out_shape=[
            jax.ShapeDtypeStruct((L, C), jnp.float32),
            jax.ShapeDtypeStruct((L, C), jnp.float32),
        ],
    )(pred2, gt2)

    out = pl.pallas_call(
        _finalize_body,
        out_shape=jax.ShapeDtypeStruct((1, 1), jnp.float32),
    )(cnt_tc, sums_tc, nsum_p.reshape(NW * L, C * LANES))
    return out[0, 0]


# speedup vs baseline: 1.1271x; 1.1271x over previous
import functools

import jax
import jax.numpy as jnp
from jax import lax
from jax.experimental import pallas as pl
from jax.experimental.pallas import tpu as pltpu
from jax.experimental.pallas import tpu_sc as plsc

NC = 2
NS = 16
LANES = 16
NW = NC * NS

L = 8
C = 16

CH = 2048
TB = 8192


def _newton_rsqrt(ss):
    i = lax.bitcast_convert_type(ss, jnp.int32)
    y = lax.bitcast_convert_type(0x5F3759DF - (i >> 1), jnp.float32)
    for _ in range(3):
        y = y * (1.5 - 0.5 * ss * y * y)
    return jnp.where(ss > 0.0, y, 0.0)


def _sc_pass(pred2, gt_flat, n_per_batch, vpw):
    k_chunks = vpw // CH
    mesh = plsc.VectorSubcoreMesh(core_axis_name="c", subcore_axis_name="s",
                                  num_cores=NC, num_subcores=NS)

    @functools.partial(
        pl.kernel,
        out_type=jax.ShapeDtypeStruct((NW, L, C * LANES), jnp.float32),
        mesh=mesh,
        compiler_params=pltpu.CompilerParams(needs_layout_passes=False),
        scratch_types=[
            pltpu.VMEM((2, C, CH), jnp.float32),
            pltpu.VMEM((2, CH), jnp.int32),
            pltpu.VMEM((L, C * LANES), jnp.float32),
            pltpu.SemaphoreType.DMA,
            pltpu.SemaphoreType.DMA,
            pltpu.SemaphoreType.DMA,
            pltpu.SemaphoreType.DMA,
        ],
    )
    def kern(pred_hbm, gt_hbm, nsum_out,
             buf, labv, nsum_t, sp0, sp1, sl0, sl1):
        cid = lax.axis_index("c")
        sid = lax.axis_index("s")
        wid = sid * NC + cid
        batch = wid // NS
        slot = wid % NS
        row0 = batch * C
        col0 = slot * vpw

        psem = (sp0, sp1)
        lsem = (sl0, sl1)

        zero16 = jnp.zeros((LANES,), jnp.float32)
        lane = lax.iota(jnp.int32, 16)
        idx1 = [lane + c * LANES for c in range(C)]

        def zinit_row(i, _):
            r = i // C
            j = i % C
            nsum_t[r, pl.ds(j * LANES, LANES)] = zero16
            return 0
        lax.fori_loop(0, L * C, zinit_row, 0)

        def chunk_coff(k):
            return col0 + k * CH

        def start(k, b):
            coff = chunk_coff(k)
            pltpu.async_copy(
                pred_hbm.at[pl.ds(row0, C), pl.ds(coff, CH)], buf.at[b],
                psem[b])
            pltpu.async_copy(
                gt_hbm.at[pl.ds(batch * n_per_batch + coff, CH)], labv.at[b],
                lsem[b])

        def wait(k, b):
            coff = chunk_coff(k)
            pltpu.make_async_copy(
                pred_hbm.at[pl.ds(row0, C), pl.ds(coff, CH)], buf.at[b],
                psem[b]).wait()
            pltpu.make_async_copy(
                gt_hbm.at[pl.ds(batch * n_per_batch + coff, CH)], labv.at[b],
                lsem[b]).wait()

        def compute(b):
            @plsc.parallel_loop(0, CH // LANES, unroll=2)
            def grp(g):
                base = g * LANES
                lv = labv[b, pl.ds(base, LANES)]
                vs = []
                sq = []
                for c in range(C):
                    v = buf[b, c, pl.ds(base, LANES)]
                    vs.append(v)
                    sq.append(v * v)
                while len(sq) > 1:
                    sq = [sq[i] + sq[i + 1] for i in range(0, len(sq), 2)]
                rinv = _newton_rsqrt(sq[0])
                for c in range(C):
                    plsc.addupdate_scatter(nsum_t, [lv, idx1[c]],
                                           vs[c] * rinv)

        start(0, 0)

        def pair_body(k2, _):
            k = k2 * 2
            start(k + 1, 1)
            wait(k, 0)
            compute(0)

            @pl.when(k2 < n_pairs - 1)
            def _():
                start(k + 2, 0)

            wait(k + 1, 1)
            compute(1)
            return 0

        n_pairs = k_chunks // 2
        lax.fori_loop(0, n_pairs, pair_body, 0)

        pltpu.sync_copy(nsum_t, nsum_out.at[wid])

    return kern(pred2, gt_flat)


def _tc_sums_body(pred_ref, gt_ref, sums_ref, cnt_ref):
    pid = pl.program_id(0)

    @pl.when(pid == 0)
    def _():
        sums_ref[...] = jnp.zeros_like(sums_ref)
        cnt_ref[...] = jnp.zeros_like(cnt_ref)

    gt_blk = gt_ref[...]
    l_iota = lax.broadcasted_iota(jnp.int32, (L, TB), 0)
    oh0 = (gt_blk[0:1, :] == l_iota).astype(jnp.float32)
    oh1 = (gt_blk[1:2, :] == l_iota).astype(jnp.float32)
    p0 = pred_ref[0:C, :]
    p1 = pred_ref[C:2 * C, :]
    dn = (((1,), (1,)), ((), ()))
    s = (lax.dot_general(oh0, p0, dn, preferred_element_type=jnp.float32)
         + lax.dot_general(oh1, p1, dn,
                           preferred_element_type=jnp.float32))
    sums_ref[...] += s
    cnt = (jnp.sum(oh0, axis=1, keepdims=True)
           + jnp.sum(oh1, axis=1, keepdims=True))
    cnt_ref[...] += jnp.broadcast_to(cnt, (L, C))


def _finalize_body(cnt_ref, sum_ref, nsum_ref, out_ref):
    rows = NW * L
    lmat = lax.broadcasted_iota(jnp.int32, (L, rows), 0)
    jmat = lax.broadcasted_iota(jnp.int32, (L, rows), 1)
    sel = (jmat % L == lmat).astype(jnp.float32)

    j2 = lax.broadcasted_iota(jnp.int32, (C * LANES, C), 0)
    c2 = lax.broadcasted_iota(jnp.int32, (C * LANES, C), 1)
    red = (j2 // LANES == c2).astype(jnp.float32)

    counts = cnt_ref[:, 0:1]
    sums = sum_ref[...]
    nsums = jnp.dot(jnp.dot(sel, nsum_ref[...],
                            preferred_element_type=jnp.float32),
                    red, preferred_element_type=jnp.float32)

    safe_c = jnp.maximum(counts, 1.0)
    means = sums / safe_c

    snorm = jnp.sqrt(jnp.sum(sums * sums, axis=1, keepdims=True))
    cos_sum = jnp.sum(sums * nsums, axis=1, keepdims=True) / jnp.maximum(
        snorm, 1e-30)
    intra_per_label = cos_sum / safe_c

    lab_idx = lax.broadcasted_iota(jnp.int32, (L, 1), 0)
    nonbg = (lab_idx > 0).astype(jnp.float32)
    intra_sim = jnp.sum(intra_per_label * nonbg, keepdims=True) / (L - 1.0)

    mnorm = jnp.sqrt(jnp.sum(means * means, axis=1, keepdims=True))
    mn = means / jnp.maximum(mnorm, 1e-8)

    total = jnp.zeros((1, 1), jnp.float32)
    for i in range(1, L - 1):
        row_i = mn[i:i + 1, :]
        simr = jnp.sum(mn * row_i, axis=1, keepdims=True)
        pair = (lab_idx > i).astype(jnp.float32)
        total = total + jnp.sum(jnp.clip(simr, 0.0, 1.0) * pair,
                                keepdims=True)
    n_pairs = (L - 1) * (L - 2) // 2
    inter = total / float(n_pairs)

    out_ref[...] = inter - intra_sim


def kernel(prediction, gt):
    b, c, z, y, x = prediction.shape
    n_per_batch = z * y * x
    n_total = b * n_per_batch
    vpw = n_per_batch // NS

    pred2 = prediction.reshape(b * c, n_per_batch)
    gt_flat = gt.reshape(n_total)
    gt2 = gt.reshape(b, n_per_batch)

    nsum_p = _sc_pass(pred2, gt_flat, n_per_batch, vpw)

    grid = n_per_batch // TB
    sums_tc, cnt_tc = pl.pallas_call(
        _tc_sums_body,
        grid=(grid,),
        in_specs=[
            pl.BlockSpec((b * c, TB), lambda i: (0, i)),
            pl.BlockSpec((b, TB), lambda i: (0, i)),
        ],
        out_specs=[
            pl.BlockSpec((L, C), lambda i: (0, 0)),
            pl.BlockSpec((L, C), lambda i: (0, 0)),
        ],
        out_shape=[
            jax.ShapeDtypeStruct((L, C), jnp.float32),
            jax.ShapeDtypeStruct((L, C), jnp.float32),
        ],
    )(pred2, gt2)

    out = pl.pallas_call(
        _finalize_body,
        out_shape=jax.ShapeDtypeStruct((1, 1), jnp.float32),
    )(cnt_tc, sums_tc, nsum_p.reshape(NW * L, C * LANES))
    return out[0, 0]
